# Initial kernel scaffold; baseline (speedup 1.0000x reference)
import jax
import jax.numpy as jnp
from jax.experimental import pallas as pl


def kernel(query, keys):
    raise NotImplementedError("write your pallas kernel here")


# speedup vs baseline: 15.5587x; 15.5587x over previous
import functools

import jax
import jax.numpy as jnp
from jax.experimental import pallas as pl
from jax.experimental.pallas import tpu as pltpu

N = 8192
M = 2048
D = 128
R = 512
NB = N // R

_HI = jax.lax.Precision.HIGHEST
_NEG = float("-inf")


def _pass1(qt_ref, keys_ref,
           ssm_ref, cm_ref, qn_ref, t1v_ref, t1i_ref, colmax_ref, colsum_ref,
           cmax_s, csum_s):
    i = pl.program_id(0)

    @pl.when(i == 0)
    def _init():
        cmax_s[...] = jnp.full((1, M), _NEG, jnp.float32)
        csum_s[...] = jnp.zeros((1, M), jnp.float32)

    qt = qt_ref[...]
    nrm = jnp.sqrt(jnp.sum(qt * qt, axis=1, keepdims=True))
    qn = qt / jnp.maximum(nrm, 1e-12)
    qn_ref[...] = qn

    keys = keys_ref[...]
    score = jax.lax.dot_general(qn, keys, (((1,), (1,)), ((), ())))

    rowmax = jnp.max(score, axis=1, keepdims=True)
    p = jnp.exp(score - rowmax)
    rowsum = jnp.sum(p, axis=1, keepdims=True)
    ssm = p / rowsum
    ssm_ref[...] = ssm
    cm_ref[...] = jax.lax.dot_general(ssm, keys, (((1,), (0,)), ((), ())))

    iota = jax.lax.broadcasted_iota(jnp.int32, (R, M), 1)
    idx1 = jnp.min(jnp.where(score == rowmax, iota, M), axis=1)
    t1v_ref[0, 0, :] = rowmax[:, 0]
    t1i_ref[0, 0, :] = idx1

    bmax = jnp.max(score, axis=0, keepdims=True)
    old_max = cmax_s[...]
    new_max = jnp.maximum(old_max, bmax)
    e = jnp.exp(score - new_max)
    csum_s[...] = (csum_s[...] * jnp.exp(old_max - new_max)
                   + jnp.sum(e, axis=0, keepdims=True))
    cmax_s[...] = new_max
    colmax_ref[...] = cmax_s[...]
    colsum_ref[...] = csum_s[...]


def _pass2(qn_ref, keys_ref, colmax_ref, colsum_ref,
           ssq_ref, um_ref, sc_ref,
           acc_s, accv_s):
    i = pl.program_id(0)

    @pl.when(i == 0)
    def _init():
        acc_s[...] = jnp.zeros((M, D), jnp.float32)
        accv_s[...] = jnp.zeros((1, 128), jnp.float32)

    qn = qn_ref[...]
    keys = keys_ref[...]
    score = jax.lax.dot_general(qn, keys, (((1,), (1,)), ((), ())))

    cmax = colmax_ref[...]
    csum = colsum_ref[...]
    e = jnp.exp(score - cmax)
    ssq_ref[...] = e / csum

    rowmax = jnp.max(score, axis=1, keepdims=True)
    iota = jax.lax.broadcasted_iota(jnp.int32, (R, M), 1)
    idx1 = jnp.min(jnp.where(score == rowmax, iota, M), axis=1)
    mask1 = iota == idx1[:, None]

    a = jnp.where(mask1, e, 0.0)
    acc_s[...] += jax.lax.dot_general(a, qn, (((0,), (0,)), ((), ())),
                                      precision=_HI)

    masked = jnp.where(mask1, _NEG, score)
    t2v = jnp.max(masked, axis=1, keepdims=True)
    idx2 = jnp.min(jnp.where(masked == t2v, iota, M), axis=1)
    mask2 = iota == idx2[:, None]

    k2 = jnp.sum(keys * keys, axis=1)[None, :]
    sk = jnp.sum(keys, axis=1)[None, :]
    qsq = jnp.sum(qn * qn, axis=1)
    sq = jnp.sum(qn, axis=1)

    k2g = jnp.sum(jnp.where(mask1, k2, 0.0), axis=1)
    skg = jnp.sum(jnp.where(mask1, sk, 0.0), axis=1)
    k2n = jnp.sum(jnp.where(mask2, k2, 0.0), axis=1)
    skn = jnp.sum(jnp.where(mask2, sk, 0.0), axis=1)

    eps = 1e-6
    deps = float(D) * eps * eps
    comp_vec = qsq + k2g - 2.0 * rowmax[:, 0]
    dp = jnp.sqrt(jnp.maximum(comp_vec + 2.0 * eps * (sq - skg) + deps, 0.0))
    dn_sq = qsq + k2n - 2.0 * t2v[:, 0] + 2.0 * eps * (sq - skn) + deps
    dn = jnp.sqrt(jnp.maximum(dn_sq, 0.0))
    sep_vec = jnp.maximum(dp - dn + 1.0, 0.0)

    lane = jax.lax.broadcasted_iota(jnp.int32, (1, 128), 1)
    accv_s[...] += (jnp.where(lane == 0, jnp.sum(sep_vec), 0.0)
                    + jnp.where(lane == 1, jnp.sum(comp_vec), 0.0))

    @pl.when(i == NB - 1)
    def _fin():
        upd = acc_s[...] + keys
        nrm = jnp.sqrt(jnp.sum(upd * upd, axis=1, keepdims=True))
        um_ref[...] = upd / jnp.maximum(nrm, 1e-12)
        scale = (jnp.where(lane == 0, 1.0 / N, 0.0)
                 + jnp.where(lane == 1, 1.0 / (N * D), 0.0))
        sc_ref[...] = accv_s[...] * scale


@jax.jit
def kernel(query, keys):
    bs, d, h, w = query.shape
    qt = jnp.transpose(query, (0, 2, 3, 1)).reshape(N, D)

    f32 = jnp.float32
    ssm, cm, qn, t1v, t1i, colmax, colsum = pl.pallas_call(
        _pass1,
        grid=(NB,),
        in_specs=[
            pl.BlockSpec((R, D), lambda i: (i, 0)),
            pl.BlockSpec((M, D), lambda i: (0, 0)),
        ],
        out_specs=[
            pl.BlockSpec((R, M), lambda i: (i, 0)),
            pl.BlockSpec((R, D), lambda i: (i, 0)),
            pl.BlockSpec((R, D), lambda i: (i, 0)),
            pl.BlockSpec((1, 1, R), lambda i: (i, 0, 0)),
            pl.BlockSpec((1, 1, R), lambda i: (i, 0, 0)),
            pl.BlockSpec((1, M), lambda i: (0, 0)),
            pl.BlockSpec((1, M), lambda i: (0, 0)),
        ],
        out_shape=[
            jax.ShapeDtypeStruct((N, M), f32),
            jax.ShapeDtypeStruct((N, D), f32),
            jax.ShapeDtypeStruct((N, D), f32),
            jax.ShapeDtypeStruct((NB, 1, R), f32),
            jax.ShapeDtypeStruct((NB, 1, R), jnp.int32),
            jax.ShapeDtypeStruct((1, M), f32),
            jax.ShapeDtypeStruct((1, M), f32),
        ],
        scratch_shapes=[
            pltpu.VMEM((1, M), f32),
            pltpu.VMEM((1, M), f32),
        ],
    )(qt, keys)

    ssq, um, sc = pl.pallas_call(
        _pass2,
        grid=(NB,),
        in_specs=[
            pl.BlockSpec((R, D), lambda i: (i, 0)),
            pl.BlockSpec((M, D), lambda i: (0, 0)),
            pl.BlockSpec((1, M), lambda i: (0, 0)),
            pl.BlockSpec((1, M), lambda i: (0, 0)),
        ],
        out_specs=[
            pl.BlockSpec((R, M), lambda i: (i, 0)),
            pl.BlockSpec((M, D), lambda i: (0, 0)),
            pl.BlockSpec((1, 128), lambda i: (0, 0)),
        ],
        out_shape=[
            jax.ShapeDtypeStruct((N, M), f32),
            jax.ShapeDtypeStruct((M, D), f32),
            jax.ShapeDtypeStruct((1, 128), f32),
        ],
        scratch_shapes=[
            pltpu.VMEM((M, D), f32),
            pltpu.VMEM((1, 128), f32),
        ],
    )(qn, keys, colmax, colsum)

    qn4 = qn.reshape(bs, h, w, D)
    cm4 = cm.reshape(bs, h, w, D)
    uq = jnp.transpose(jnp.concatenate([qn4, cm4], axis=3), (0, 3, 1, 2))

    return uq, um, ssq, ssm, sc[0, 0], sc[0, 1]


# speedup vs baseline: 18.4404x; 1.1852x over previous
import functools

import jax
from jax import lax
import jax.numpy as jnp
from jax.experimental import pallas as pl
from jax.experimental.pallas import tpu as pltpu
from jax.experimental.pallas import tpu_sc as plsc

N = 8192
M = 2048
D = 128
R = 512
NB = N // R

NC = 2
NS = 16
NW = NC * NS
CHUNK = N // NW
ZR = M // NS

_HI = jax.lax.Precision.HIGHEST
_NEG = float("-inf")


def _pass1(qt_ref, keys_ref,
           ssm_ref, cm_ref, qn_ref, t1v_ref, t1i_ref, colmax_ref, colsum_ref,
           cmax_s, csum_s):
    i = pl.program_id(0)

    @pl.when(i == 0)
    def _init():
        cmax_s[...] = jnp.full((1, M), _NEG, jnp.float32)
        csum_s[...] = jnp.zeros((1, M), jnp.float32)

    qt = qt_ref[...]
    nrm = jnp.sqrt(jnp.sum(qt * qt, axis=1, keepdims=True))
    qn = qt / jnp.maximum(nrm, 1e-12)
    qn_ref[...] = qn

    keys = keys_ref[...]
    score = jax.lax.dot_general(qn, keys, (((1,), (1,)), ((), ())))

    rowmax = jnp.max(score, axis=1, keepdims=True)
    p = jnp.exp(score - rowmax)
    rowsum = jnp.sum(p, axis=1, keepdims=True)
    ssm = p / rowsum
    ssm_ref[...] = ssm
    cm_ref[...] = jax.lax.dot_general(ssm, keys, (((1,), (0,)), ((), ())))

    iota = jax.lax.broadcasted_iota(jnp.int32, (R, M), 1)
    idx1 = jnp.min(jnp.where(score == rowmax, iota, M), axis=1)
    t1v_ref[0, 0, :] = rowmax[:, 0]
    t1i_ref[0, 0, :] = idx1

    bmax = jnp.max(score, axis=0, keepdims=True)
    old_max = cmax_s[...]
    new_max = jnp.maximum(old_max, bmax)
    e = jnp.exp(score - new_max)
    csum_s[...] = (csum_s[...] * jnp.exp(old_max - new_max)
                   + jnp.sum(e, axis=0, keepdims=True))
    cmax_s[...] = new_max
    colmax_ref[...] = cmax_s[...]
    colsum_ref[...] = csum_s[...]


def _pass2(qn_ref, keys_ref, colmax_ref, colsum_ref,
           ssq_ref, sc_ref,
           accv_s):
    i = pl.program_id(0)

    @pl.when(i == 0)
    def _init():
        accv_s[...] = jnp.zeros((1, 128), jnp.float32)

    qn = qn_ref[...]
    keys = keys_ref[...]
    score = jax.lax.dot_general(qn, keys, (((1,), (1,)), ((), ())))

    cmax = colmax_ref[...]
    csum = colsum_ref[...]
    e = jnp.exp(score - cmax)
    ssq_ref[...] = e / csum

    rowmax = jnp.max(score, axis=1, keepdims=True)
    iota = jax.lax.broadcasted_iota(jnp.int32, (R, M), 1)
    idx1 = jnp.min(jnp.where(score == rowmax, iota, M), axis=1)
    mask1 = iota == idx1[:, None]

    masked = jnp.where(mask1, _NEG, score)
    t2v = jnp.max(masked, axis=1, keepdims=True)
    idx2 = jnp.min(jnp.where(masked == t2v, iota, M), axis=1)
    mask2 = iota == idx2[:, None]

    k2 = jnp.sum(keys * keys, axis=1)[None, :]
    sk = jnp.sum(keys, axis=1)[None, :]
    qsq = jnp.sum(qn * qn, axis=1)
    sq = jnp.sum(qn, axis=1)

    k2g = jnp.sum(jnp.where(mask1, k2, 0.0), axis=1)
    skg = jnp.sum(jnp.where(mask1, sk, 0.0), axis=1)
    k2n = jnp.sum(jnp.where(mask2, k2, 0.0), axis=1)
    skn = jnp.sum(jnp.where(mask2, sk, 0.0), axis=1)

    eps = 1e-6
    deps = float(D) * eps * eps
    comp_vec = qsq + k2g - 2.0 * rowmax[:, 0]
    dp = jnp.sqrt(jnp.maximum(comp_vec + 2.0 * eps * (sq - skg) + deps, 0.0))
    dn_sq = qsq + k2n - 2.0 * t2v[:, 0] + 2.0 * eps * (sq - skn) + deps
    dn = jnp.sqrt(jnp.maximum(dn_sq, 0.0))
    sep_vec = jnp.maximum(dp - dn + 1.0, 0.0)

    lane = jax.lax.broadcasted_iota(jnp.int32, (1, 128), 1)
    accv_s[...] += (jnp.where(lane == 0, jnp.sum(sep_vec), 0.0)
                    + jnp.where(lane == 1, jnp.sum(comp_vec), 0.0))

    @pl.when(i == NB - 1)
    def _fin():
        scale = (jnp.where(lane == 0, 1.0 / N, 0.0)
                 + jnp.where(lane == 1, 1.0 / (N * D), 0.0))
        sc_ref[...] = accv_s[...] * scale


def _sc_scatter(qn_hbm, g_hbm, t1v_hbm, cmax_hbm, out_hbm,
                qn_v, g_v, t1v_v, cm_v, w_v, s_v, shared, sem):
    c = lax.axis_index("c")
    s = lax.axis_index("s")
    wid = s * NC + c
    base = wid * CHUNK

    pltpu.sync_copy(qn_hbm.at[pl.ds(base, CHUNK)], qn_v)
    pltpu.sync_copy(g_hbm.at[pl.ds(wid * 2, 2)], g_v)
    pltpu.sync_copy(t1v_hbm.at[pl.ds(base, CHUNK)], t1v_v)
    for j in range(2):
        pltpu.async_copy(cmax_hbm.at[g_v.at[j]], cm_v.at[j], sem).wait()

    def _zbody(j, carry):
        for t in range(D // 16):
            s_v[j, pl.ds(t * 16, 16)] = jnp.zeros((16,), jnp.float32)
        return carry

    lax.fori_loop(0, ZR, _zbody, 0)
    pltpu.sync_copy(s_v.at[pl.ds(0, ZR)], shared.at[pl.ds(s * ZR, ZR)])

    for t in range(CHUNK // 16):
        j, off = divmod(t * 16, 128)
        w_v[j, pl.ds(off, 16)] = jnp.exp(t1v_v[pl.ds(t * 16, 16)]
                                         - cm_v[j, pl.ds(off, 16)])

    def _sbody(rg, carry):
        w16 = w_v[rg >> 3, pl.ds((rg & 7) * 16, 16)]
        for rr in range(16):
            r = rg * 16 + rr
            wsp = jnp.zeros((16,), jnp.float32) + w16[rr]
            for t in range(D // 16):
                s_v[r, pl.ds(t * 16, 16)] = qn_v[r, pl.ds(t * 16, 16)] * wsp
        return carry

    lax.fori_loop(0, CHUNK // 16, _sbody, 0)

    plsc.subcore_barrier()
    for j in range(2):
        pltpu.sync_copy(s_v.at[pl.ds(j * 128, 128)],
                        shared.at[g_v.at[j]], add=True)
    plsc.subcore_barrier()
    pltpu.sync_copy(shared.at[pl.ds(s * ZR, ZR)],
                    out_hbm.at[c, pl.ds(s * ZR, ZR)])


def _pass3(parts_ref, keys_ref, um_ref):
    upd = parts_ref[0] + parts_ref[1] + keys_ref[...]
    nrm = jnp.sqrt(jnp.sum(upd * upd, axis=1, keepdims=True))
    um_ref[...] = upd / jnp.maximum(nrm, 1e-12)


@jax.jit
def kernel(query, keys):
    bs, d, h, w = query.shape
    qt = jnp.transpose(query, (0, 2, 3, 1)).reshape(N, D)

    f32 = jnp.float32
    ssm, cm, qn, t1v, t1i, colmax, colsum = pl.pallas_call(
        _pass1,
        grid=(NB,),
        in_specs=[
            pl.BlockSpec((R, D), lambda i: (i, 0)),
            pl.BlockSpec((M, D), lambda i: (0, 0)),
        ],
        out_specs=[
            pl.BlockSpec((R, M), lambda i: (i, 0)),
            pl.BlockSpec((R, D), lambda i: (i, 0)),
            pl.BlockSpec((R, D), lambda i: (i, 0)),
            pl.BlockSpec((1, 1, R), lambda i: (i, 0, 0)),
            pl.BlockSpec((1, 1, R), lambda i: (i, 0, 0)),
            pl.BlockSpec((1, M), lambda i: (0, 0)),
            pl.BlockSpec((1, M), lambda i: (0, 0)),
        ],
        out_shape=[
            jax.ShapeDtypeStruct((N, M), f32),
            jax.ShapeDtypeStruct((N, D), f32),
            jax.ShapeDtypeStruct((N, D), f32),
            jax.ShapeDtypeStruct((NB, 1, R), f32),
            jax.ShapeDtypeStruct((NB, 1, R), jnp.int32),
            jax.ShapeDtypeStruct((1, M), f32),
            jax.ShapeDtypeStruct((1, M), f32),
        ],
        scratch_shapes=[
            pltpu.VMEM((1, M), f32),
            pltpu.VMEM((1, M), f32),
        ],
    )(qt, keys)

    ssq, sc = pl.pallas_call(
        _pass2,
        grid=(NB,),
        in_specs=[
            pl.BlockSpec((R, D), lambda i: (i, 0)),
            pl.BlockSpec((M, D), lambda i: (0, 0)),
            pl.BlockSpec((1, M), lambda i: (0, 0)),
            pl.BlockSpec((1, M), lambda i: (0, 0)),
        ],
        out_specs=[
            pl.BlockSpec((R, M), lambda i: (i, 0)),
            pl.BlockSpec((1, 128), lambda i: (0, 0)),
        ],
        out_shape=[
            jax.ShapeDtypeStruct((N, M), f32),
            jax.ShapeDtypeStruct((1, 128), f32),
        ],
        scratch_shapes=[
            pltpu.VMEM((1, 128), f32),
        ],
    )(qn, keys, colmax, colsum)

    sc_call = pl.kernel(
        _sc_scatter,
        mesh=plsc.VectorSubcoreMesh(core_axis_name="c", subcore_axis_name="s"),
        out_type=jax.ShapeDtypeStruct((NC, M, D), f32),
        scratch_types=[
            pltpu.VMEM((CHUNK, D), f32),
            pltpu.VMEM((2, 128), jnp.int32),
            pltpu.VMEM((CHUNK,), f32),
            pltpu.VMEM((2, 128), f32),
            pltpu.VMEM((2, 128), f32),
            pltpu.VMEM((CHUNK, D), f32),
            pltpu.VMEM_SHARED((M, D), f32),
            pltpu.SemaphoreType.DMA,
        ],
    )
    parts = sc_call(qn, t1i.reshape(N // 128, 128), t1v.reshape(N),
                    colmax.reshape(M))

    um = pl.pallas_call(
        _pass3,
        out_shape=jax.ShapeDtypeStruct((M, D), f32),
    )(parts, keys)

    qn4 = qn.reshape(bs, h, w, D)
    cm4 = cm.reshape(bs, h, w, D)
    uq = jnp.transpose(jnp.concatenate([qn4, cm4], axis=3), (0, 3, 1, 2))

    return uq, um, ssq, ssm, sc[0, 0], sc[0, 1]
